# baseline (speedup 1.0000x reference)
import jax
import jax.numpy as jnp
from jax import lax
from jax.experimental import pallas as pl
from jax.experimental.pallas import tpu as pltpu
from jax.experimental.pallas import tpu_sc as plsc

_BATCH = 16384
_N_FIELDS = 26
_EMBED_DIM = 16
_R = _BATCH * _N_FIELDS
_NW = 32
_SPB = 128
_NBLOCKS = _R // _SPB
_BW = _NBLOCKS // _NW
_CBLK = 8
_NCHUNK = _BW // _CBLK
_CROWS = _CBLK * _SPB
_NBB = _BATCH // _SPB

_C0 = 1.1447097560713972e-05
_C1 = 0.9991664010110775
_C2 = -0.48969909032091086
_C3 = 0.28382318306553606
_C4 = -0.1299571976585037
_C5 = 0.0298087652435521


def _softplus16(v):
    e = jnp.exp(-jnp.abs(v))
    p = jnp.float32(_C5)
    for c in (_C4, _C3, _C2, _C1, _C0):
        p = p * e + jnp.float32(c)
    return jnp.maximum(v, jnp.float32(0.0)) + p


def _body(x_hbm, params_hbm, out_hbm, idx_v, g_v, p_v, t_v, gsem, osem):
    wid = lax.axis_index("s") * 2 + lax.axis_index("c")
    t0 = wid * _BW

    pltpu.sync_copy(x_hbm.at[pl.ds(t0, _BW)], idx_v)

    lanes = lax.iota(jnp.int32, 16)

    def gathers(g, buf):
        return [
            pltpu.make_async_copy(
                params_hbm.at[idx_v.at[g * _CBLK + j]],
                g_v.at[buf].at[pl.ds(j * _SPB, _SPB)],
                gsem.at[buf],
            )
            for j in range(_CBLK)
        ]

    def out_copies(g, buf):
        tbase = t0 + g * _CBLK
        f = tbase // _NBB
        c0 = tbase % _NBB
        return [
            pltpu.make_async_copy(
                t_v.at[buf].at[pl.ds(r * (_CBLK * 1024), _CBLK * 1024)],
                out_hbm.at[pl.ds((f * 256 + r * 128 + c0) * 1024, _CBLK * 1024)],
                osem.at[buf],
            )
            for r in range(2)
        ]

    for c in gathers(0, 0):
        c.start()

    for g in range(_NCHUNK):
        buf = g % 2
        if g + 1 < _NCHUNK:
            for c in gathers(g + 1, 1 - buf):
                c.start()
        for c in gathers(g, buf):
            c.wait()
        if g >= 2:
            for c in out_copies(g - 2, buf):
                c.wait()

        @plsc.parallel_loop(0, _CROWS, unroll=4)
        def softplus_rows(r):
            p_v[r, pl.ds(0, _EMBED_DIM)] = _softplus16(g_v[buf, r, :])

        @plsc.parallel_loop(0, 128, unroll=2)
        def transpose(j):
            r = j >> 6
            s = (j >> 3) & 7
            l0 = (j & 7) << 4
            d = (r << 3) | s
            dsel = jnp.full((16,), d, jnp.int32)
            rbase = lanes + l0
            tbase = (r << 13) + (s << 7) + l0
            for b in range(_CBLK):
                v = plsc.load_gather(p_v, [rbase + (b * _SPB), dsel])
                t_v[buf, pl.ds(tbase + (b << 10), 16)] = v
        for c in out_copies(g, buf):
            c.start()

    for c in out_copies(_NCHUNK - 2, (_NCHUNK - 2) % 2):
        c.wait()
    for c in out_copies(_NCHUNK - 1, (_NCHUNK - 1) % 2):
        c.wait()


_K1ROWS = 2048
_K1FULL = 488


def _detile_chunk(p_hbm, out_hbm, in_v, p17_v, lin_v, lanes, start, nrows):
    pltpu.sync_copy(
        p_hbm.at[:, pl.ds(start, nrows)], in_v.at[:, pl.ds(0, nrows)]
    )

    gshift = (nrows // 16).bit_length() - 1
    gmask = (nrows // 16) - 1

    @plsc.parallel_loop(0, _EMBED_DIM * (nrows // 16), unroll=4)
    def _stage(m):
        d = m >> gshift
        l0 = (m & gmask) << 4
        p17_v[d, pl.ds(l0, 16)] = in_v[d, pl.ds(l0, 16)]

    @plsc.parallel_loop(0, nrows, unroll=4)
    def _rows(r):
        v = plsc.load_gather(p17_v, [lanes, jnp.full((16,), r, jnp.int32)])
        lin_v[pl.ds(r * _EMBED_DIM, _EMBED_DIM)] = v

    pltpu.sync_copy(
        lin_v.at[pl.ds(0, nrows * _EMBED_DIM)],
        out_hbm.at[pl.ds(start * _EMBED_DIM, nrows * _EMBED_DIM)],
    )


def _detile_body(p_hbm, tail_hbm, out_hbm, in_v, p17_v, in_t, lin_v):
    wid = lax.axis_index("s") * 2 + lax.axis_index("c")
    lanes = lax.iota(jnp.int32, 16)
    for k in range(16):
        c = wid + 32 * k

        @pl.when(c < _K1FULL)
        def _():
            _detile_chunk(
                p_hbm, out_hbm, in_v, p17_v, lin_v, lanes, c * _K1ROWS, _K1ROWS
            )

    @pl.when(wid == 0)
    def _():
        _detile_chunk(p_hbm, out_hbm, in_v, p17_v, lin_v, lanes, 999424, 512)

    @pl.when(wid == 1)
    def _():
        pltpu.sync_copy(tail_hbm, in_t)

        @plsc.parallel_loop(0, 64, unroll=4)
        def _rows(r):
            v = plsc.load_gather(in_t, [lanes, jnp.full((16,), r, jnp.int32)])
            lin_v[pl.ds(r * _EMBED_DIM, _EMBED_DIM)] = v

        pltpu.sync_copy(
            lin_v.at[pl.ds(0, 64 * _EMBED_DIM)],
            out_hbm.at[pl.ds(999936 * _EMBED_DIM, 64 * _EMBED_DIM)],
        )


_mesh = plsc.VectorSubcoreMesh(core_axis_name="c", subcore_axis_name="s")

_detile = pl.kernel(
    _detile_body,
    out_type=jax.ShapeDtypeStruct((1000000 * _EMBED_DIM,), jnp.float32),
    mesh=_mesh,
    scratch_types=[
        pltpu.VMEM((_EMBED_DIM, _K1ROWS), jnp.float32),
        pltpu.VMEM((_EMBED_DIM, _K1ROWS + 1), jnp.float32),
        pltpu.VMEM((_EMBED_DIM, 64), jnp.float32),
        pltpu.VMEM((_K1ROWS * _EMBED_DIM,), jnp.float32),
    ],
    compiler_params=pltpu.CompilerParams(
        use_tc_tiling_on_sc=True, needs_layout_passes=False
    ),
)

_gather_softplus = pl.kernel(
    _body,
    out_type=jax.ShapeDtypeStruct((_R * _EMBED_DIM,), jnp.float32),
    mesh=_mesh,
    scratch_types=[
        pltpu.VMEM((_BW, _SPB), jnp.int32),
        pltpu.VMEM((2, _CROWS, _EMBED_DIM), jnp.float32),
        pltpu.VMEM((_CROWS, _EMBED_DIM + 1), jnp.float32),
        pltpu.VMEM((2, 2 * _CBLK * 1024), jnp.float32),
        pltpu.SemaphoreType.DMA((2,)),
        pltpu.SemaphoreType.DMA((2,)),
    ],
    compiler_params=pltpu.CompilerParams(
        use_tc_tiling_on_sc=False, needs_layout_passes=False
    ),
)


def kernel(x, params):
    xt = x.astype(jnp.int32).T.reshape(_NBLOCKS, _SPB)
    pt = params.T
    plin = _detile(pt, pt[:, 999936:]).reshape(1000000, _EMBED_DIM)
    out = _gather_softplus(xt, plin)
    return (
        out.reshape(_N_FIELDS, 2, _NBB, 8, _SPB)
        .transpose(2, 4, 0, 1, 3)
        .reshape(_BATCH, _N_FIELDS, _EMBED_DIM)
    )


# speedup vs baseline: 1.2073x; 1.2073x over previous
import jax
import jax.numpy as jnp
from jax import lax
from jax.experimental import pallas as pl
from jax.experimental.pallas import tpu as pltpu
from jax.experimental.pallas import tpu_sc as plsc

_BATCH = 16384
_N_FIELDS = 26
_EMBED_DIM = 16
_R = _BATCH * _N_FIELDS
_NW = 32
_SPB = 128
_NBLOCKS = _R // _SPB
_BW = _NBLOCKS // _NW
_CBLK = 8
_NCHUNK = _BW // _CBLK
_CROWS = _CBLK * _SPB
_NBB = _BATCH // _SPB

_C0 = 1.1447097560713972e-05
_C1 = 0.9991664010110775
_C2 = -0.48969909032091086
_C3 = 0.28382318306553606
_C4 = -0.1299571976585037
_C5 = 0.0298087652435521


def _softplus16(v):
    e = jnp.exp(-jnp.abs(v))
    p = jnp.float32(_C5)
    for c in (_C4, _C3, _C2, _C1, _C0):
        p = p * e + jnp.float32(c)
    return jnp.maximum(v, jnp.float32(0.0)) + p


def _body(x_hbm, params_hbm, out_hbm, idx_v, g_v, p_v, t_v, gsem, osem):
    wid = lax.axis_index("s") * 2 + lax.axis_index("c")
    t0 = wid * _BW

    pltpu.sync_copy(x_hbm.at[pl.ds(t0, _BW)], idx_v)

    lanes = lax.iota(jnp.int32, 16)

    def gathers(g, buf):
        return [
            pltpu.make_async_copy(
                params_hbm.at[idx_v.at[g * _CBLK + j]],
                g_v.at[buf].at[pl.ds(j * _SPB, _SPB)],
                gsem.at[buf],
            )
            for j in range(_CBLK)
        ]

    def out_copies(g, buf):
        tbase = t0 + g * _CBLK
        f = tbase // _NBB
        c0 = tbase % _NBB
        return [
            pltpu.make_async_copy(
                t_v.at[buf].at[pl.ds(r * (_CBLK * 1024), _CBLK * 1024)],
                out_hbm.at[pl.ds((f * 256 + r * 128 + c0) * 1024, _CBLK * 1024)],
                osem.at[buf],
            )
            for r in range(2)
        ]

    for c in gathers(0, 0):
        c.start()

    for g in range(_NCHUNK):
        buf = g % 2
        if g + 1 < _NCHUNK:
            for c in gathers(g + 1, 1 - buf):
                c.start()
        for c in gathers(g, buf):
            c.wait()
        if g >= 2:
            for c in out_copies(g - 2, buf):
                c.wait()

        @plsc.parallel_loop(0, _CROWS, unroll=4)
        def softplus_rows(r):
            p_v[r, pl.ds(0, _EMBED_DIM)] = _softplus16(g_v[buf, r, :])

        @plsc.parallel_loop(0, 128, unroll=2)
        def transpose(j):
            r = j >> 6
            s = (j >> 3) & 7
            l0 = (j & 7) << 4
            d = (r << 3) | s
            dsel = jnp.full((16,), d, jnp.int32)
            rbase = lanes + l0
            tbase = (r << 13) + (s << 7) + l0
            for b in range(_CBLK):
                v = plsc.load_gather(p_v, [rbase + (b * _SPB), dsel])
                t_v[buf, pl.ds(tbase + (b << 10), 16)] = v
        for c in out_copies(g, buf):
            c.start()

    for c in out_copies(_NCHUNK - 2, (_NCHUNK - 2) % 2):
        c.wait()
    for c in out_copies(_NCHUNK - 1, (_NCHUNK - 1) % 2):
        c.wait()


_K1ROWS = 1024
_K1FULL = 976
_K1K = 31


def _detile_body(p_hbm, tail_hbm, s_hbm, out_hbm, in_v, in_s, in_t, lin_v,
                 isem, osem):
    wid = lax.axis_index("s") * 2 + lax.axis_index("c")
    lanes = lax.iota(jnp.int32, 16)

    def in_copy(c, buf):
        return pltpu.make_async_copy(
            p_hbm.at[:, pl.ds(c * _K1ROWS, _K1ROWS)], in_v.at[buf],
            isem.at[buf],
        )

    def out_copy(c, buf):
        return pltpu.make_async_copy(
            lin_v.at[buf],
            out_hbm.at[pl.ds(c * (_K1ROWS * _EMBED_DIM), _K1ROWS * _EMBED_DIM)],
            osem.at[buf],
        )

    def compute(src, buf, nrows):
        @plsc.parallel_loop(0, nrows, unroll=4)
        def _rows(r):
            v = plsc.load_gather(src, [lanes, jnp.full((16,), r, jnp.int32)])
            lin_v[buf, pl.ds(r * _EMBED_DIM, _EMBED_DIM)] = v

    in_copy(wid, 0).start()
    for k in range(_K1K):
        c = wid + 32 * k
        buf = k % 2

        def step(k=k, c=c, buf=buf):
            if k + 1 < _K1K - 1:
                in_copy(c + 32, 1 - buf).start()
            elif k + 1 == _K1K - 1:
                @pl.when(wid < _K1FULL - 32 * (_K1K - 1))
                def _():
                    in_copy(c + 32, 1 - buf).start()
            in_copy(c, buf).wait()
            if k >= 2:
                out_copy(c - 64, buf).wait()
            compute(in_v.at[buf], buf, _K1ROWS)
            out_copy(c, buf).start()

        if k < _K1K - 1:
            step()
        else:
            @pl.when(wid < _K1FULL - 32 * (_K1K - 1))
            def _():
                step()

    nlast = _K1FULL - 32 * (_K1K - 1)

    @pl.when(wid < nlast)
    def _():
        out_copy(wid + 32 * (_K1K - 1), (_K1K - 1) % 2).wait()
        out_copy(wid + 32 * (_K1K - 2), (_K1K - 2) % 2).wait()

    @pl.when(wid >= nlast)
    def _():
        out_copy(wid + 32 * (_K1K - 2), (_K1K - 2) % 2).wait()
        out_copy(wid + 32 * (_K1K - 3), (_K1K - 3) % 2).wait()

    @pl.when(wid == 0)
    def _():
        pltpu.sync_copy(s_hbm, in_s)
        compute(in_s, 0, 512)
        pltpu.sync_copy(
            lin_v.at[0].at[pl.ds(0, 512 * _EMBED_DIM)],
            out_hbm.at[pl.ds(999424 * _EMBED_DIM, 512 * _EMBED_DIM)],
        )

    @pl.when(wid == 1)
    def _():
        pltpu.sync_copy(tail_hbm, in_t)
        compute(in_t, 0, 64)
        pltpu.sync_copy(
            lin_v.at[0].at[pl.ds(0, 64 * _EMBED_DIM)],
            out_hbm.at[pl.ds(999936 * _EMBED_DIM, 64 * _EMBED_DIM)],
        )


_mesh = plsc.VectorSubcoreMesh(core_axis_name="c", subcore_axis_name="s")

_detile = pl.kernel(
    _detile_body,
    out_type=jax.ShapeDtypeStruct((1000000 * _EMBED_DIM,), jnp.float32),
    mesh=_mesh,
    scratch_types=[
        pltpu.VMEM((2, _EMBED_DIM, _K1ROWS), jnp.float32),
        pltpu.VMEM((_EMBED_DIM, 512), jnp.float32),
        pltpu.VMEM((_EMBED_DIM, 64), jnp.float32),
        pltpu.VMEM((2, _K1ROWS * _EMBED_DIM), jnp.float32),
        pltpu.SemaphoreType.DMA((2,)),
        pltpu.SemaphoreType.DMA((2,)),
    ],
    compiler_params=pltpu.CompilerParams(
        use_tc_tiling_on_sc=True, needs_layout_passes=False
    ),
)

_gather_softplus = pl.kernel(
    _body,
    out_type=jax.ShapeDtypeStruct((_R * _EMBED_DIM,), jnp.float32),
    mesh=_mesh,
    scratch_types=[
        pltpu.VMEM((_BW, _SPB), jnp.int32),
        pltpu.VMEM((2, _CROWS, _EMBED_DIM), jnp.float32),
        pltpu.VMEM((_CROWS, _EMBED_DIM + 1), jnp.float32),
        pltpu.VMEM((2, 2 * _CBLK * 1024), jnp.float32),
        pltpu.SemaphoreType.DMA((2,)),
        pltpu.SemaphoreType.DMA((2,)),
    ],
    compiler_params=pltpu.CompilerParams(
        use_tc_tiling_on_sc=False, needs_layout_passes=False
    ),
)


def kernel(x, params):
    xt = x.astype(jnp.int32).T.reshape(_NBLOCKS, _SPB)
    pt = params.T
    plin = _detile(pt, pt[:, 999936:], pt[:, 999424:999936]).reshape(
        1000000, _EMBED_DIM
    )
    out = _gather_softplus(xt, plin)
    return (
        out.reshape(_N_FIELDS, 2, _NBB, 8, _SPB)
        .transpose(2, 4, 0, 1, 3)
        .reshape(_BATCH, _N_FIELDS, _EMBED_DIM)
    )
